# Initial kernel scaffold; baseline (speedup 1.0000x reference)
import jax
import jax.numpy as jnp
from jax.experimental import pallas as pl


def kernel(x, edge_index, W, b):
    raise NotImplementedError("write your pallas kernel here")


# speedup vs baseline: 14.2083x; 14.2083x over previous
import functools

import jax
import jax.numpy as jnp
from jax import lax
from jax.experimental import pallas as pl
from jax.experimental.pallas import tpu as pltpu
from jax.experimental.pallas import tpu_sc as plsc

N = 10000
F = 128
E = 320000

NC = 2
NS = 16
NW = NC * NS
K = 128
EPT = E // NW
NCH = 80
EPT_PAD = NCH * K
NPAD = 10240
RPT = NPAD // NS
DEGW = 16

_MESH = plsc.VectorSubcoreMesh(
    core_axis_name="c", subcore_axis_name="s", num_cores=NC, num_subcores=NS)


def _deg_body(dst_hbm, out_hbm, dst_v, buf, degsh):
    cid = lax.axis_index("c")
    sid = lax.axis_index("s")
    wid = sid * NC + cid

    def fill(val):
        def row(r, c):
            buf[r, :] = jnp.full((DEGW,), val, jnp.float32)
            return c
        lax.fori_loop(0, K, row, 0)

    fill(0.0)
    for j in range(RPT // K):
        pltpu.sync_copy(buf, degsh.at[pl.ds(sid * RPT + j * K, K)])
    fill(1.0)
    pltpu.sync_copy(dst_hbm.at[wid], dst_v)
    plsc.subcore_barrier()

    def chunk(i, c):
        pltpu.sync_copy(buf, degsh.at[dst_v.at[i]], add=True)
        return c
    lax.fori_loop(0, NCH, chunk, 0)
    plsc.subcore_barrier()
    pltpu.sync_copy(degsh.at[pl.ds(sid * RPT, RPT)],
                    out_hbm.at[cid, pl.ds(sid * RPT, RPT)])


_deg_call = functools.partial(
    pl.kernel,
    out_type=jax.ShapeDtypeStruct((NC, NPAD, DEGW), jnp.float32),
    mesh=_MESH,
    scratch_types=[
        pltpu.VMEM((NCH, K), jnp.int32),
        pltpu.VMEM((K, DEGW), jnp.float32),
        pltpu.VMEM_SHARED((NPAD, DEGW), jnp.float32),
    ],
)(_deg_body)


CHG = 16


def _gs_body(src_hbm, dst_hbm, y_hbm, out_hbm, src_g, dst_g, bufa, bufb,
             gsh, sema, semb):
    cid = lax.axis_index("c")
    sid = lax.axis_index("s")
    wid = sid * NC + cid

    def zrow(r, c):
        for j in range(F // 16):
            bufa[r, pl.ds(j * 16, 16)] = jnp.zeros((16,), jnp.float32)
        return c
    lax.fori_loop(0, K, zrow, 0)
    for j in range(RPT // K):
        pltpu.sync_copy(bufa, gsh.at[pl.ds(sid * RPT + j * K, K)])
    plsc.subcore_barrier()

    def group(gi, c):
        pltpu.sync_copy(src_hbm.at[wid, pl.ds(gi * CHG, CHG)], src_g)
        pltpu.sync_copy(dst_hbm.at[wid, pl.ds(gi * CHG, CHG)], dst_g)
        pltpu.async_copy(y_hbm.at[src_g.at[0]], bufa, sema)

        def step2(t, c2):
            i = t * 2
            pltpu.async_copy(y_hbm.at[src_g.at[i + 1]], bufb, semb)
            pltpu.make_async_copy(y_hbm.at[src_g.at[i]], bufa, sema).wait()
            pltpu.sync_copy(bufa, gsh.at[dst_g.at[i]], add=True)

            @pl.when(i + 2 < CHG)
            def _():
                pltpu.async_copy(y_hbm.at[src_g.at[i + 2]], bufa, sema)
            pltpu.make_async_copy(y_hbm.at[src_g.at[i + 1]], bufb, semb).wait()
            pltpu.sync_copy(bufb, gsh.at[dst_g.at[i + 1]], add=True)
            return c2
        lax.fori_loop(0, CHG // 2, step2, 0)
        return c
    lax.fori_loop(0, NCH // CHG, group, 0)
    plsc.subcore_barrier()
    pltpu.sync_copy(gsh.at[pl.ds(sid * RPT, RPT)],
                    out_hbm.at[cid, pl.ds(sid * RPT, RPT)])


_gs_call = functools.partial(
    pl.kernel,
    out_type=jax.ShapeDtypeStruct((NC, NPAD, F), jnp.float32),
    mesh=_MESH,
    scratch_types=[
        pltpu.VMEM((CHG, K), jnp.int32),
        pltpu.VMEM((CHG, K), jnp.int32),
        pltpu.VMEM((K, F), jnp.float32),
        pltpu.VMEM((K, F), jnp.float32),
        pltpu.VMEM_SHARED((NPAD, F), jnp.float32),
        pltpu.SemaphoreType.DMA,
        pltpu.SemaphoreType.DMA,
    ],
)(_gs_body)


_BR = 1024


def _mm_body(x_ref, wt_ref, degc_ref, y_ref):
    deg = degc_ref[0, :] + degc_ref[1, :] + 1.0
    dinv = lax.rsqrt(deg)
    y = jnp.dot(x_ref[...], wt_ref[...],
                preferred_element_type=jnp.float32,
                precision=lax.Precision.HIGHEST)
    y_ref[...] = y * dinv[:, None]


def _mm_call(x, wt, degc):
    return pl.pallas_call(
        _mm_body,
        grid=(NPAD // _BR,),
        in_specs=[
            pl.BlockSpec((_BR, F), lambda i: (i, 0)),
            pl.BlockSpec((F, F), lambda i: (0, 0)),
            pl.BlockSpec((NC, _BR), lambda i: (0, i)),
        ],
        out_specs=pl.BlockSpec((_BR, F), lambda i: (i, 0)),
        out_shape=jax.ShapeDtypeStruct((NPAD, F), jnp.float32),
    )(x, wt, degc)


_DR = 1024


def _fin_body(x_ref, y_ref, g_ref, degc_ref, b_ref, o_ref):
    deg = degc_ref[0, :] + degc_ref[1, :] + 1.0
    dinv = lax.rsqrt(deg)
    gs = g_ref[0] + g_ref[1] + y_ref[...]
    o_ref[...] = x_ref[...] + jnp.maximum(
        gs * dinv[:, None] + b_ref[...][None, :], 0.0)


def _fin_call(x, y, g, degc, b):
    return pl.pallas_call(
        _fin_body,
        grid=(pl.cdiv(N, _DR),),
        in_specs=[
            pl.BlockSpec((_DR, F), lambda i: (i, 0)),
            pl.BlockSpec((_DR, F), lambda i: (i, 0)),
            pl.BlockSpec((NC, _DR, F), lambda i: (0, i, 0)),
            pl.BlockSpec((NC, _DR), lambda i: (0, i)),
            pl.BlockSpec((F,), lambda i: (0,)),
        ],
        out_specs=pl.BlockSpec((_DR, F), lambda i: (i, 0)),
        out_shape=jax.ShapeDtypeStruct((N, F), jnp.float32),
    )(x, y, g, degc, b)


def kernel(x, edge_index, W, b):
    src = edge_index[0].astype(jnp.int32)
    dst = edge_index[1].astype(jnp.int32)
    pad = jnp.full((NW, EPT_PAD - EPT), N, jnp.int32)
    srcp = jnp.concatenate([src.reshape(NW, EPT), pad], axis=1)
    srcp = srcp.reshape(NW, NCH, K)
    dstp = jnp.concatenate([dst.reshape(NW, EPT), pad], axis=1)
    dstp = dstp.reshape(NW, NCH, K)

    degp = _deg_call(dstp)
    degc = degp[:, :, 0]
    y = _mm_call(x, W.T, degc)
    g = _gs_call(srcp, dstp, y)
    return _fin_call(x, y, g, degc, b)


# speedup vs baseline: 14.2909x; 1.0058x over previous
import functools

import jax
import jax.numpy as jnp
from jax import lax
from jax.experimental import pallas as pl
from jax.experimental.pallas import tpu as pltpu
from jax.experimental.pallas import tpu_sc as plsc

N = 10000
F = 128
E = 320000

NC = 2
NS = 16
NW = NC * NS
K = 128
EPT = E // NW
NCH = 80
EPT_PAD = NCH * K
NPAD = 10240
RPT = NPAD // NS
DEGW = 16

_MESH = plsc.VectorSubcoreMesh(
    core_axis_name="c", subcore_axis_name="s", num_cores=NC, num_subcores=NS)


DEG_FK = 16


def _deg_body(dst_hbm, out_hbm, dst_v, buf, degsh, sem):
    cid = lax.axis_index("c")
    sid = lax.axis_index("s")
    wid = sid * NC + cid

    def fill(val):
        def row(r, c):
            buf[r, :] = jnp.full((DEGW,), val, jnp.float32)
            return c
        lax.fori_loop(0, K, row, 0)

    fill(0.0)
    for j in range(RPT // K):
        pltpu.sync_copy(buf, degsh.at[pl.ds(sid * RPT + j * K, K)])
    fill(1.0)
    pltpu.sync_copy(dst_hbm.at[wid], dst_v)
    plsc.subcore_barrier()

    def group(g, c):
        def fire(i, c2):
            pltpu.async_copy(buf, degsh.at[dst_v.at[g * DEG_FK + i]], sem,
                             add=True)
            return c2
        lax.fori_loop(0, DEG_FK, fire, 0)

        def drain(i, c2):
            pltpu.make_async_copy(buf, degsh.at[dst_v.at[0]], sem).wait()
            return c2
        lax.fori_loop(0, DEG_FK, drain, 0)
        return c
    lax.fori_loop(0, NCH // DEG_FK, group, 0)
    plsc.subcore_barrier()
    pltpu.sync_copy(degsh.at[pl.ds(sid * RPT, RPT)],
                    out_hbm.at[cid, pl.ds(sid * RPT, RPT)])


_deg_call = functools.partial(
    pl.kernel,
    out_type=jax.ShapeDtypeStruct((NC, NPAD, DEGW), jnp.float32),
    mesh=_MESH,
    scratch_types=[
        pltpu.VMEM((NCH, K), jnp.int32),
        pltpu.VMEM((K, DEGW), jnp.float32),
        pltpu.VMEM_SHARED((NPAD, DEGW), jnp.float32),
        pltpu.SemaphoreType.DMA,
    ],
)(_deg_body)


CHG = 16


def _gs_body(src_hbm, dst_hbm, y_hbm, out_hbm, src_g, dst_g, bufa, bufb,
             gsh, sema, semb):
    cid = lax.axis_index("c")
    sid = lax.axis_index("s")
    wid = sid * NC + cid

    def zrow(r, c):
        for j in range(F // 16):
            bufa[r, pl.ds(j * 16, 16)] = jnp.zeros((16,), jnp.float32)
        return c
    lax.fori_loop(0, K, zrow, 0)
    for j in range(RPT // K):
        pltpu.sync_copy(bufa, gsh.at[pl.ds(sid * RPT + j * K, K)])
    plsc.subcore_barrier()

    def group(gi, c):
        pltpu.sync_copy(src_hbm.at[wid, pl.ds(gi * CHG, CHG)], src_g)
        pltpu.sync_copy(dst_hbm.at[wid, pl.ds(gi * CHG, CHG)], dst_g)
        pltpu.async_copy(y_hbm.at[src_g.at[0]], bufa, sema)

        def step2(t, c2):
            i = t * 2
            pltpu.async_copy(y_hbm.at[src_g.at[i + 1]], bufb, semb)
            pltpu.make_async_copy(y_hbm.at[src_g.at[i]], bufa, sema).wait()
            pltpu.sync_copy(bufa, gsh.at[dst_g.at[i]], add=True)

            @pl.when(i + 2 < CHG)
            def _():
                pltpu.async_copy(y_hbm.at[src_g.at[i + 2]], bufa, sema)
            pltpu.make_async_copy(y_hbm.at[src_g.at[i + 1]], bufb, semb).wait()
            pltpu.sync_copy(bufb, gsh.at[dst_g.at[i + 1]], add=True)
            return c2
        lax.fori_loop(0, CHG // 2, step2, 0)
        return c
    lax.fori_loop(0, NCH // CHG, group, 0)
    plsc.subcore_barrier()
    pltpu.sync_copy(gsh.at[pl.ds(sid * RPT, RPT)],
                    out_hbm.at[cid, pl.ds(sid * RPT, RPT)])


_gs_call = functools.partial(
    pl.kernel,
    out_type=jax.ShapeDtypeStruct((NC, NPAD, F), jnp.float32),
    mesh=_MESH,
    scratch_types=[
        pltpu.VMEM((CHG, K), jnp.int32),
        pltpu.VMEM((CHG, K), jnp.int32),
        pltpu.VMEM((K, F), jnp.float32),
        pltpu.VMEM((K, F), jnp.float32),
        pltpu.VMEM_SHARED((NPAD, F), jnp.float32),
        pltpu.SemaphoreType.DMA,
        pltpu.SemaphoreType.DMA,
    ],
)(_gs_body)


_BR = 1024


def _mm_body(x_ref, wt_ref, degc_ref, y_ref):
    deg = degc_ref[0, :] + degc_ref[1, :] + 1.0
    dinv = lax.rsqrt(deg)
    y = jnp.dot(x_ref[...], wt_ref[...],
                preferred_element_type=jnp.float32,
                precision=lax.Precision.HIGHEST)
    y_ref[...] = y * dinv[:, None]


def _mm_call(x, wt, degc):
    return pl.pallas_call(
        _mm_body,
        grid=(NPAD // _BR,),
        in_specs=[
            pl.BlockSpec((_BR, F), lambda i: (i, 0)),
            pl.BlockSpec((F, F), lambda i: (0, 0)),
            pl.BlockSpec((NC, _BR), lambda i: (0, i)),
        ],
        out_specs=pl.BlockSpec((_BR, F), lambda i: (i, 0)),
        out_shape=jax.ShapeDtypeStruct((NPAD, F), jnp.float32),
    )(x, wt, degc)


_DR = 1024


def _fin_body(x_ref, y_ref, g_ref, degc_ref, b_ref, o_ref):
    deg = degc_ref[0, :] + degc_ref[1, :] + 1.0
    dinv = lax.rsqrt(deg)
    gs = g_ref[0] + g_ref[1] + y_ref[...]
    o_ref[...] = x_ref[...] + jnp.maximum(
        gs * dinv[:, None] + b_ref[...][None, :], 0.0)


def _fin_call(x, y, g, degc, b):
    return pl.pallas_call(
        _fin_body,
        grid=(pl.cdiv(N, _DR),),
        in_specs=[
            pl.BlockSpec((_DR, F), lambda i: (i, 0)),
            pl.BlockSpec((_DR, F), lambda i: (i, 0)),
            pl.BlockSpec((NC, _DR, F), lambda i: (0, i, 0)),
            pl.BlockSpec((NC, _DR), lambda i: (0, i)),
            pl.BlockSpec((F,), lambda i: (0,)),
        ],
        out_specs=pl.BlockSpec((_DR, F), lambda i: (i, 0)),
        out_shape=jax.ShapeDtypeStruct((N, F), jnp.float32),
    )(x, y, g, degc, b)


def kernel(x, edge_index, W, b):
    src = edge_index[0].astype(jnp.int32)
    dst = edge_index[1].astype(jnp.int32)
    pad = jnp.full((NW, EPT_PAD - EPT), N, jnp.int32)
    srcp = jnp.concatenate([src.reshape(NW, EPT), pad], axis=1)
    srcp = srcp.reshape(NW, NCH, K)
    dstp = jnp.concatenate([dst.reshape(NW, EPT), pad], axis=1)
    dstp = dstp.reshape(NW, NCH, K)

    degp = _deg_call(dstp)
    degc = degp[:, :, 0]
    y = _mm_call(x, W.T, degc)
    g = _gs_call(srcp, dstp, y)
    return _fin_call(x, y, g, degc, b)
